# Initial kernel scaffold; baseline (speedup 1.0000x reference)
import jax
import jax.numpy as jnp
from jax.experimental import pallas as pl


def kernel(x, edge_index, W_self1, W_neigh1, b1, W_self2, W_neigh2, b2, W_self3, W_neigh3, b3):
    raise NotImplementedError("write your pallas kernel here")


# speedup vs baseline: 4.6265x; 4.6265x over previous
import functools

import jax
import jax.numpy as jnp
from jax import lax
from jax.experimental import pallas as pl
from jax.experimental.pallas import tpu as pltpu
from jax.experimental.pallas import tpu_sc as plsc

N = 10000
E = 320000
D = 128
H = 128
C = 32

NPAD = 10240
NC = 2
NS = 16
NW = NC * NS
EPW = E // NW
K = 80
NCHUNK = EPW // K
RPT = NPAD // NS

BR = 1024


def _make_sc_agg(d, with_deg):
  mesh = plsc.VectorSubcoreMesh(core_axis_name="c", subcore_axis_name="s")

  out_type = [jax.ShapeDtypeStruct((NC, NPAD, d), jnp.float32)]
  scratch = [
      pltpu.VMEM((K,), jnp.int32),
      pltpu.VMEM((K,), jnp.int32),
      pltpu.VMEM((K, d), jnp.float32),
      pltpu.VMEM_SHARED((NPAD, d), jnp.float32),
      pltpu.SemaphoreType.DMA,
  ]
  if with_deg:
    out_type.append(jax.ShapeDtypeStruct((NC, NPAD), jnp.float32))
    scratch += [
        pltpu.VMEM((K,), jnp.float32),
        pltpu.VMEM_SHARED((NPAD,), jnp.float32),
    ]

  if with_deg:
    @functools.partial(pl.kernel, mesh=mesh, out_type=out_type,
                       scratch_types=scratch)
    def k(src_hbm, dst_hbm, h_hbm, zrows_hbm, zvec_hbm, ones_hbm,
          out_hbm, deg_hbm,
          srcv, dstv, rowsv, acc, sem, onesv, dacc):
      cid = lax.axis_index("c")
      sid = lax.axis_index("s")
      pltpu.sync_copy(zrows_hbm, acc.at[pl.ds(sid * RPT, RPT)])
      pltpu.sync_copy(ones_hbm, onesv)
      pltpu.sync_copy(zvec_hbm, dacc.at[pl.ds(sid * RPT, RPT)])
      plsc.subcore_barrier()
      base = (cid * NS + sid) * EPW

      def chunk(i, carry):
        off = base + i * K
        pltpu.sync_copy(src_hbm.at[pl.ds(off, K)], srcv)
        pltpu.sync_copy(dst_hbm.at[pl.ds(off, K)], dstv)
        pltpu.async_copy(h_hbm.at[srcv], rowsv, sem).wait()
        pltpu.sync_copy(rowsv, acc.at[dstv], add=True)
        pltpu.sync_copy(onesv, dacc.at[dstv], add=True)
        return carry

      lax.fori_loop(0, NCHUNK, chunk, 0)
      plsc.subcore_barrier()
      pltpu.sync_copy(acc.at[pl.ds(sid * RPT, RPT)],
                      out_hbm.at[cid, pl.ds(sid * RPT, RPT)])
      pltpu.sync_copy(dacc.at[pl.ds(sid * RPT, RPT)],
                      deg_hbm.at[cid, pl.ds(sid * RPT, RPT)])
  else:
    @functools.partial(pl.kernel, mesh=mesh, out_type=out_type,
                       scratch_types=scratch)
    def k(src_hbm, dst_hbm, h_hbm, zrows_hbm,
          out_hbm,
          srcv, dstv, rowsv, acc, sem):
      cid = lax.axis_index("c")
      sid = lax.axis_index("s")
      pltpu.sync_copy(zrows_hbm, acc.at[pl.ds(sid * RPT, RPT)])
      plsc.subcore_barrier()
      base = (cid * NS + sid) * EPW

      def chunk(i, carry):
        off = base + i * K
        pltpu.sync_copy(src_hbm.at[pl.ds(off, K)], srcv)
        pltpu.sync_copy(dst_hbm.at[pl.ds(off, K)], dstv)
        pltpu.async_copy(h_hbm.at[srcv], rowsv, sem).wait()
        pltpu.sync_copy(rowsv, acc.at[dstv], add=True)
        return carry

      lax.fori_loop(0, NCHUNK, chunk, 0)
      plsc.subcore_barrier()
      pltpu.sync_copy(acc.at[pl.ds(sid * RPT, RPT)],
                      out_hbm.at[cid, pl.ds(sid * RPT, RPT)])

  return k


_sc_agg_deg = _make_sc_agg(D, with_deg=True)
_sc_agg_h = _make_sc_agg(H, with_deg=False)


def _row_spec(d):
  return pl.BlockSpec((BR, d), lambda i: (i, 0))


def _full_spec(shape):
  nd = len(shape)
  return pl.BlockSpec(shape, lambda i: (0,) * nd)


def _dense1_body(h_ref, a0_ref, a1_ref, d0_ref, d1_ref, ws_ref, wn_ref,
                 b_ref, o_ref):
  inv = 1.0 / jnp.maximum(d0_ref[...] + d1_ref[...], 1.0)
  mean = (a0_ref[...] + a1_ref[...]) * inv
  acc = jnp.dot(h_ref[...], ws_ref[...], preferred_element_type=jnp.float32)
  acc += jnp.dot(mean, wn_ref[...], preferred_element_type=jnp.float32)
  acc += b_ref[...]
  acc = jnp.maximum(acc, 0.0)
  nrm = jnp.sqrt(jnp.sum(acc * acc, axis=1, keepdims=True))
  o_ref[...] = acc / (nrm + 1e-12)


def _dense1(h, a0, a1, d0, d1, ws, wn, b):
  return pl.pallas_call(
      _dense1_body,
      grid=(NPAD // BR,),
      in_specs=[
          _row_spec(D), _row_spec(D), _row_spec(D),
          _row_spec(1), _row_spec(1),
          _full_spec((D, H)), _full_spec((D, H)), _full_spec((1, H)),
      ],
      out_specs=_row_spec(H),
      out_shape=jax.ShapeDtypeStruct((NPAD, H), jnp.float32),
  )(h, a0, a1, d0, d1, ws, wn, b)


def _dense2_body(h_ref, a0_ref, a1_ref, d0_ref, d1_ref, ws_ref, wn_ref,
                 b_ref, ws3_ref, b3_ref, hs_ref, hn_ref):
  inv = 1.0 / jnp.maximum(d0_ref[...] + d1_ref[...], 1.0)
  mean = (a0_ref[...] + a1_ref[...]) * inv
  acc = jnp.dot(h_ref[...], ws_ref[...], preferred_element_type=jnp.float32)
  acc += jnp.dot(mean, wn_ref[...], preferred_element_type=jnp.float32)
  acc += b_ref[...]
  acc = jnp.maximum(acc, 0.0)
  nrm = jnp.sqrt(jnp.sum(acc * acc, axis=1, keepdims=True))
  h2 = acc / (nrm + 1e-12)
  hs_ref[...] = (jnp.dot(h2, ws3_ref[...], preferred_element_type=jnp.float32)
                 + b3_ref[...])
  hn_ref[...] = h2


def _dense2(h, a0, a1, d0, d1, ws, wn, b, ws3, b3):
  return pl.pallas_call(
      _dense2_body,
      grid=(NPAD // BR,),
      in_specs=[
          _row_spec(H), _row_spec(H), _row_spec(H),
          _row_spec(1), _row_spec(1),
          _full_spec((H, H)), _full_spec((H, H)), _full_spec((1, H)),
          _full_spec((H, C)), _full_spec((1, C)),
      ],
      out_specs=[_row_spec(C), _row_spec(H)],
      out_shape=[jax.ShapeDtypeStruct((NPAD, C), jnp.float32),
                 jax.ShapeDtypeStruct((NPAD, H), jnp.float32)],
  )(h, a0, a1, d0, d1, ws, wn, b, ws3, b3)


def _post3_body(hs_ref, a0_ref, a1_ref, d0_ref, d1_ref, wn3_ref, o_ref):
  inv = 1.0 / jnp.maximum(d0_ref[...] + d1_ref[...], 1.0)
  mean = (a0_ref[...] + a1_ref[...]) * inv
  z = hs_ref[...] + jnp.dot(mean, wn3_ref[...],
                            preferred_element_type=jnp.float32)
  m = jnp.max(z, axis=1, keepdims=True)
  e = jnp.exp(z - m)
  o_ref[...] = e / jnp.sum(e, axis=1, keepdims=True)


def _post3(hs, a0, a1, d0, d1, wn3):
  return pl.pallas_call(
      _post3_body,
      grid=(NPAD // BR,),
      in_specs=[
          _row_spec(C), _row_spec(H), _row_spec(H),
          _row_spec(1), _row_spec(1),
          _full_spec((H, C)),
      ],
      out_specs=_row_spec(C),
      out_shape=jax.ShapeDtypeStruct((NPAD, C), jnp.float32),
  )(hs, a0, a1, d0, d1, wn3)


def kernel(x, edge_index, W_self1, W_neigh1, b1, W_self2, W_neigh2, b2,
           W_self3, W_neigh3, b3):
  src = edge_index[0]
  dst = edge_index[1]
  x_pad = jnp.pad(x, ((0, NPAD - N), (0, 0)))

  zrows = jnp.zeros((RPT, D), jnp.float32)
  zvec = jnp.zeros((RPT,), jnp.float32)
  ones = jnp.ones((K,), jnp.float32)

  agg1, deg = _sc_agg_deg(src, dst, x_pad, zrows, zvec, ones)
  d0 = deg[0].reshape(NPAD, 1)
  d1 = deg[1].reshape(NPAD, 1)

  h1 = _dense1(x_pad, agg1[0], agg1[1], d0, d1,
               W_self1, W_neigh1, b1.reshape(1, H))

  agg2, = _sc_agg_h(src, dst, h1, zrows)
  hs3, h2 = _dense2(h1, agg2[0], agg2[1], d0, d1,
                    W_self2, W_neigh2, b2.reshape(1, H),
                    W_self3, b3.reshape(1, C))

  agg3, = _sc_agg_h(src, dst, h2, zrows)
  out = _post3(hs3, agg3[0], agg3[1], d0, d1, W_neigh3)
  return out[:N]


# speedup vs baseline: 8.1728x; 1.7665x over previous
import functools

import jax
import jax.numpy as jnp
from jax import lax
from jax.experimental import pallas as pl
from jax.experimental.pallas import tpu as pltpu
from jax.experimental.pallas import tpu_sc as plsc

N = 10000
E = 320000
D = 128
H = 128
C = 32

NPAD = 10240
NC = 2
NS = 16
NW = NC * NS
EPW = E // NW
K = 80
NCHUNK = EPW // K
IDXSHIFT = 14
RPT = NPAD // NS

BR = 1024


def _make_sc_agg(d, with_deg):
  mesh = plsc.VectorSubcoreMesh(core_axis_name="c", subcore_axis_name="s")

  out_type = [jax.ShapeDtypeStruct((NC, NPAD, d), jnp.float32)]
  scratch = [
      pltpu.VMEM((NCHUNK, K), jnp.int32),
      pltpu.VMEM((K,), jnp.int32),
      pltpu.VMEM((K,), jnp.int32),
      pltpu.VMEM((K,), jnp.int32),
      pltpu.VMEM((K,), jnp.int32),
      pltpu.VMEM((K, d), jnp.float32),
      pltpu.VMEM((K, d), jnp.float32),
      pltpu.VMEM_SHARED((NPAD, d), jnp.float32),
      pltpu.SemaphoreType.DMA,
      pltpu.SemaphoreType.DMA,
  ]
  if with_deg:
    out_type.append(jax.ShapeDtypeStruct((NC, NPAD), jnp.float32))
    scratch += [
        pltpu.VMEM((K,), jnp.float32),
        pltpu.VMEM_SHARED((NPAD,), jnp.float32),
    ]

  NPAIR = (NCHUNK - 1) // 2

  def _pipeline(h_hbm, packedall, srcb, dstb, rows, sems, acc,
                deg_scatter):
    def unpack(i, b):
      for j in range(K // 16):
        v = packedall[i, pl.ds(j * 16, 16)]
        srcb[b][pl.ds(j * 16, 16)] = lax.bitwise_and(v, (1 << IDXSHIFT) - 1)
        dstb[b][pl.ds(j * 16, 16)] = lax.shift_right_logical(v, IDXSHIFT)

    def gather_start(b):
      pltpu.make_async_copy(h_hbm.at[srcb[b]], rows[b], sems[b]).start()

    def gather_wait(b):
      pltpu.make_async_copy(h_hbm.at[srcb[b]], rows[b], sems[b]).wait()

    def scatter(b):
      pltpu.sync_copy(rows[b], acc.at[dstb[b]], add=True)
      deg_scatter(dstb[b])

    unpack(0, 0)
    gather_start(0)

    def pair(p, carry):
      i = 2 * p
      gather_wait(0)
      unpack(i + 1, 1)
      gather_start(1)
      scatter(0)
      gather_wait(1)
      unpack(i + 2, 0)
      gather_start(0)
      scatter(1)
      return carry

    lax.fori_loop(0, NPAIR, pair, 0)
    for t in range(2 * NPAIR, NCHUNK):
      b = t % 2
      gather_wait(b)
      if t + 1 < NCHUNK:
        unpack(t + 1, 1 - b)
        gather_start(1 - b)
      scatter(b)

  if with_deg:
    @functools.partial(pl.kernel, mesh=mesh, out_type=out_type,
                       scratch_types=scratch)
    def k(pk_hbm, h_hbm, zrows_hbm, zvec_hbm, ones_hbm,
          out_hbm, deg_hbm,
          packedall, src0, src1, dst0, dst1, rows0, rows1, acc,
          sem0, sem1, onesv, dacc):
      cid = lax.axis_index("c")
      sid = lax.axis_index("s")
      wid = cid * NS + sid
      pltpu.sync_copy(zrows_hbm, acc.at[pl.ds(sid * RPT, RPT)])
      pltpu.sync_copy(ones_hbm, onesv)
      pltpu.sync_copy(zvec_hbm, dacc.at[pl.ds(sid * RPT, RPT)])
      pltpu.sync_copy(pk_hbm.at[wid], packedall)
      plsc.subcore_barrier()

      def deg_scatter(dref):
        pltpu.sync_copy(onesv, dacc.at[dref], add=True)

      _pipeline(h_hbm, packedall, (src0, src1), (dst0, dst1),
                (rows0, rows1), (sem0, sem1), acc, deg_scatter)
      plsc.subcore_barrier()
      pltpu.sync_copy(acc.at[pl.ds(sid * RPT, RPT)],
                      out_hbm.at[cid, pl.ds(sid * RPT, RPT)])
      pltpu.sync_copy(dacc.at[pl.ds(sid * RPT, RPT)],
                      deg_hbm.at[cid, pl.ds(sid * RPT, RPT)])
  else:
    @functools.partial(pl.kernel, mesh=mesh, out_type=out_type,
                       scratch_types=scratch)
    def k(pk_hbm, h_hbm, zrows_hbm,
          out_hbm,
          packedall, src0, src1, dst0, dst1, rows0, rows1, acc,
          sem0, sem1):
      cid = lax.axis_index("c")
      sid = lax.axis_index("s")
      wid = cid * NS + sid
      pltpu.sync_copy(zrows_hbm, acc.at[pl.ds(sid * RPT, RPT)])
      pltpu.sync_copy(pk_hbm.at[wid], packedall)
      plsc.subcore_barrier()
      _pipeline(h_hbm, packedall, (src0, src1), (dst0, dst1),
                (rows0, rows1), (sem0, sem1), acc, lambda dref: None)
      plsc.subcore_barrier()
      pltpu.sync_copy(acc.at[pl.ds(sid * RPT, RPT)],
                      out_hbm.at[cid, pl.ds(sid * RPT, RPT)])

  return k


_sc_agg_deg = _make_sc_agg(D, with_deg=True)
_sc_agg_h = _make_sc_agg(H, with_deg=False)


def _row_spec(d):
  return pl.BlockSpec((BR, d), lambda i: (i, 0))


def _full_spec(shape):
  nd = len(shape)
  return pl.BlockSpec(shape, lambda i: (0,) * nd)


def _dense1_body(h_ref, a0_ref, a1_ref, d0_ref, d1_ref, ws_ref, wn_ref,
                 b_ref, o_ref):
  inv = 1.0 / jnp.maximum(d0_ref[...] + d1_ref[...], 1.0)
  mean = (a0_ref[...] + a1_ref[...]) * inv
  acc = jnp.dot(h_ref[...], ws_ref[...], preferred_element_type=jnp.float32)
  acc += jnp.dot(mean, wn_ref[...], preferred_element_type=jnp.float32)
  acc += b_ref[...]
  acc = jnp.maximum(acc, 0.0)
  nrm = jnp.sqrt(jnp.sum(acc * acc, axis=1, keepdims=True))
  o_ref[...] = acc / (nrm + 1e-12)


def _dense1(h, a0, a1, d0, d1, ws, wn, b):
  return pl.pallas_call(
      _dense1_body,
      grid=(NPAD // BR,),
      in_specs=[
          _row_spec(D), _row_spec(D), _row_spec(D),
          _row_spec(1), _row_spec(1),
          _full_spec((D, H)), _full_spec((D, H)), _full_spec((1, H)),
      ],
      out_specs=_row_spec(H),
      out_shape=jax.ShapeDtypeStruct((NPAD, H), jnp.float32),
  )(h, a0, a1, d0, d1, ws, wn, b)


def _dense2_body(h_ref, a0_ref, a1_ref, d0_ref, d1_ref, ws_ref, wn_ref,
                 b_ref, ws3_ref, b3_ref, hs_ref, hn_ref):
  inv = 1.0 / jnp.maximum(d0_ref[...] + d1_ref[...], 1.0)
  mean = (a0_ref[...] + a1_ref[...]) * inv
  acc = jnp.dot(h_ref[...], ws_ref[...], preferred_element_type=jnp.float32)
  acc += jnp.dot(mean, wn_ref[...], preferred_element_type=jnp.float32)
  acc += b_ref[...]
  acc = jnp.maximum(acc, 0.0)
  nrm = jnp.sqrt(jnp.sum(acc * acc, axis=1, keepdims=True))
  h2 = acc / (nrm + 1e-12)
  hs_ref[...] = (jnp.dot(h2, ws3_ref[...], preferred_element_type=jnp.float32)
                 + b3_ref[...])
  hn_ref[...] = h2


def _dense2(h, a0, a1, d0, d1, ws, wn, b, ws3, b3):
  return pl.pallas_call(
      _dense2_body,
      grid=(NPAD // BR,),
      in_specs=[
          _row_spec(H), _row_spec(H), _row_spec(H),
          _row_spec(1), _row_spec(1),
          _full_spec((H, H)), _full_spec((H, H)), _full_spec((1, H)),
          _full_spec((H, C)), _full_spec((1, C)),
      ],
      out_specs=[_row_spec(C), _row_spec(H)],
      out_shape=[jax.ShapeDtypeStruct((NPAD, C), jnp.float32),
                 jax.ShapeDtypeStruct((NPAD, H), jnp.float32)],
  )(h, a0, a1, d0, d1, ws, wn, b, ws3, b3)


def _post3_body(hs_ref, a0_ref, a1_ref, d0_ref, d1_ref, wn3_ref, o_ref):
  inv = 1.0 / jnp.maximum(d0_ref[...] + d1_ref[...], 1.0)
  mean = (a0_ref[...] + a1_ref[...]) * inv
  z = hs_ref[...] + jnp.dot(mean, wn3_ref[...],
                            preferred_element_type=jnp.float32)
  m = jnp.max(z, axis=1, keepdims=True)
  e = jnp.exp(z - m)
  o_ref[...] = e / jnp.sum(e, axis=1, keepdims=True)


def _post3(hs, a0, a1, d0, d1, wn3):
  return pl.pallas_call(
      _post3_body,
      grid=(NPAD // BR,),
      in_specs=[
          _row_spec(C), _row_spec(H), _row_spec(H),
          _row_spec(1), _row_spec(1),
          _full_spec((H, C)),
      ],
      out_specs=_row_spec(C),
      out_shape=jax.ShapeDtypeStruct((NPAD, C), jnp.float32),
  )(hs, a0, a1, d0, d1, wn3)


def kernel(x, edge_index, W_self1, W_neigh1, b1, W_self2, W_neigh2, b2,
           W_self3, W_neigh3, b3):
  packed = (edge_index[0] + (edge_index[1] << IDXSHIFT)).reshape(
      NW, NCHUNK, K)
  x_pad = jnp.pad(x, ((0, NPAD - N), (0, 0)))

  zrows = jnp.zeros((RPT, D), jnp.float32)
  zvec = jnp.zeros((RPT,), jnp.float32)
  ones = jnp.ones((K,), jnp.float32)

  agg1, deg = _sc_agg_deg(packed, x_pad, zrows, zvec, ones)
  d0 = deg[0].reshape(NPAD, 1)
  d1 = deg[1].reshape(NPAD, 1)

  h1 = _dense1(x_pad, agg1[0], agg1[1], d0, d1,
               W_self1, W_neigh1, b1.reshape(1, H))

  agg2, = _sc_agg_h(packed, h1, zrows)
  hs3, h2 = _dense2(h1, agg2[0], agg2[1], d0, d1,
                    W_self2, W_neigh2, b2.reshape(1, H),
                    W_self3, b3.reshape(1, C))

  agg3, = _sc_agg_h(packed, h2, zrows)
  out = _post3(hs3, agg3[0], agg3[1], d0, d1, W_neigh3)
  return out[:N]
